# baseline (speedup 1.0000x reference)
import jax
import jax.numpy as jnp
from jax import lax
from jax.experimental import pallas as pl
from jax.experimental.pallas import tpu as pltpu
from jax.experimental.pallas import tpu_sc as plsc

N_NODES = 10000
N_PAD = 10240
DUMP = 10000
NC, NS = 2, 16
NW = NC * NS
B = 64
SA = 160
SB = 160
SEG_STEPS = 32
SEG_ALL = SB // SEG_STEPS
SEG_A = SA // SEG_STEPS
NBUF = 4
LOOK = 2
DEGW = 16
ROWS_PER_TILE = N_PAD // NS
RB = 1024
GRID = N_PAD // RB


def _sc_mesh():
    return plsc.VectorSubcoreMesh(core_axis_name="c", subcore_axis_name="s")


def _deg_body(dst_hbm, deg_out, idx_v, ones_v, zb_v, deg_sh, dsem):
    cid = lax.axis_index("c")
    sid = lax.axis_index("s")
    wid = cid * NS + sid
    row0 = sid * ROWS_PER_TILE

    def fill_ones(t, c):
        ones_v[t, :] = jnp.full((DEGW,), 1.0, jnp.float32)
        return c

    lax.fori_loop(0, B, fill_ones, 0)

    def fill_z(t, c):
        zb_v[t, :] = jnp.zeros((DEGW,), jnp.float32)
        return c

    lax.fori_loop(0, 16, fill_z, 0)

    def zero_cp(t, c):
        pltpu.sync_copy(zb_v, deg_sh.at[pl.ds(row0 + t * 16, 16)])
        return c

    lax.fori_loop(0, ROWS_PER_TILE // 16, zero_cp, 0)
    plsc.subcore_barrier()

    pltpu.sync_copy(dst_hbm.at[wid], idx_v)

    def batch(tb, c):
        def fire(j, c2):
            pltpu.async_copy(ones_v, deg_sh.at[idx_v.at[tb * 8 + j]], dsem,
                             add=True)
            return c2

        lax.fori_loop(0, 8, fire, 0)

        def drain(j, c2):
            pltpu.make_async_copy(ones_v, deg_sh.at[idx_v.at[tb * 8 + j]],
                                  dsem).wait()
            return c2

        lax.fori_loop(0, 8, drain, 0)
        return c

    nb = jnp.where(cid == 0, SA // 8, SB // 8)
    lax.fori_loop(0, nb, batch, 0)
    plsc.subcore_barrier()

    pltpu.sync_copy(deg_sh.at[pl.ds(row0, ROWS_PER_TILE)],
                    deg_out.at[cid, pl.ds(row0, ROWS_PER_TILE)])


def _deg_call(dst_p):
    return pl.kernel(
        _deg_body,
        out_type=jax.ShapeDtypeStruct((NC, N_PAD, DEGW), jnp.float32),
        mesh=_sc_mesh(),
        scratch_types=[
            pltpu.VMEM((SA, B), jnp.int32),
            pltpu.VMEM((B, DEGW), jnp.float32),
            pltpu.VMEM((16, DEGW), jnp.float32),
            pltpu.VMEM_SHARED((N_PAD, DEGW), jnp.float32),
            pltpu.SemaphoreType.DMA,
        ],
    )(dst_p)


def _edge_body(src_hbm, dst_hbm, g_hbm, acc_out,
               src_v, dst_v, rows_v, zb_v, acc_sh,
               gsem0, gsem1, gsem2, gsem3, ssem0, ssem1, ssem2, ssem3):
    gsem = (gsem0, gsem1, gsem2, gsem3)
    ssem = (ssem0, ssem1, ssem2, ssem3)
    cid = lax.axis_index("c")
    sid = lax.axis_index("s")
    wid = cid * NS + sid
    row0 = sid * ROWS_PER_TILE

    def fill_z(t, c):
        r = t // 8
        col = t % 8
        zb_v[r, pl.ds(col * 16, 16)] = jnp.zeros((16,), jnp.float32)
        return c

    lax.fori_loop(0, 64, fill_z, 0)

    def zero_cp(t, c):
        pltpu.sync_copy(zb_v, acc_sh.at[pl.ds(row0 + t * 8, 8)])
        return c

    lax.fori_loop(0, ROWS_PER_TILE // 8, zero_cp, 0)
    plsc.subcore_barrier()

    def run_segment(s, active):
        def guard(extra, fn):
            def run():
                fn()

            if active is None and extra is None:
                run()
            elif active is None:
                pl.when(extra)(run)
            elif extra is None:
                pl.when(active)(run)
            else:
                pl.when(jnp.logical_and(active, extra))(run)

        guard(None, lambda: pltpu.sync_copy(
            src_hbm.at[wid, pl.ds(s * SEG_STEPS, SEG_STEPS)], src_v))
        guard(None, lambda: pltpu.sync_copy(
            dst_hbm.at[wid, pl.ds(s * SEG_STEPS, SEG_STEPS)], dst_v))

        for b in range(LOOK):
            guard(None, lambda b=b: pltpu.async_copy(
                g_hbm.at[src_v.at[b]], rows_v.at[b], gsem[b]))

        def outer(t, c):
            j0 = t * NBUF
            for b in range(NBUF):
                j = j0 + b
                b2 = (b + LOOK) % NBUF
                guard(None, lambda b=b, j=j: pltpu.make_async_copy(
                    g_hbm.at[src_v.at[j]], rows_v.at[b], gsem[b]).wait())
                guard(None, lambda b=b, j=j: pltpu.async_copy(
                    rows_v.at[b], acc_sh.at[dst_v.at[j]], ssem[b], add=True))
                guard(j >= LOOK, lambda b2=b2, j=j: pltpu.make_async_copy(
                    rows_v.at[b2], acc_sh.at[dst_v.at[j - LOOK]],
                    ssem[b2]).wait())
                guard(j + LOOK < SEG_STEPS, lambda b2=b2, j=j:
                      pltpu.async_copy(g_hbm.at[src_v.at[j + LOOK]],
                                       rows_v.at[b2], gsem[b2]))
            return c

        lax.fori_loop(0, SEG_STEPS // NBUF, outer, 0)
        for j in range(SEG_STEPS - LOOK, SEG_STEPS):
            guard(None, lambda j=j: pltpu.make_async_copy(
                rows_v.at[j % NBUF], acc_sh.at[dst_v.at[j]],
                ssem[j % NBUF]).wait())

    for s in range(SEG_A):
        run_segment(s, None if s < SEG_ALL else (cid == 0))
    plsc.subcore_barrier()

    pltpu.sync_copy(acc_sh.at[pl.ds(row0, ROWS_PER_TILE)],
                    acc_out.at[cid, pl.ds(row0, ROWS_PER_TILE)])


def _edge_call(src_p, dst_p, g):
    h = g.shape[1]
    return pl.kernel(
        _edge_body,
        out_type=jax.ShapeDtypeStruct((NC, N_PAD, h), jnp.float32),
        mesh=_sc_mesh(),
        scratch_types=[
            pltpu.VMEM((SEG_STEPS, B), jnp.int32),
            pltpu.VMEM((SEG_STEPS, B), jnp.int32),
            pltpu.VMEM((NBUF, B, h), jnp.float32),
            pltpu.VMEM((8, h), jnp.float32),
            pltpu.VMEM_SHARED((N_PAD, h), jnp.float32),
            pltpu.SemaphoreType.DMA,
            pltpu.SemaphoreType.DMA,
            pltpu.SemaphoreType.DMA,
            pltpu.SemaphoreType.DMA,
            pltpu.SemaphoreType.DMA,
            pltpu.SemaphoreType.DMA,
            pltpu.SemaphoreType.DMA,
            pltpu.SemaphoreType.DMA,
        ],
    )(src_p, dst_p, g)


def _mm_body(x_ref, w_ref, deg_ref, g_ref):
    deg = deg_ref[0, :, 0] + deg_ref[1, :, 0] + 1.0
    dis = lax.rsqrt(deg)
    h = jnp.dot(x_ref[...], w_ref[...], preferred_element_type=jnp.float32)
    g_ref[...] = h * dis[:, None]


def _mm_call(x_pad, w, deg2):
    c = x_pad.shape[1]
    h = w.shape[1]
    return pl.pallas_call(
        _mm_body,
        grid=(GRID,),
        in_specs=[
            pl.BlockSpec((RB, c), lambda i: (i, 0)),
            pl.BlockSpec((c, h), lambda i: (0, 0)),
            pl.BlockSpec((NC, RB, DEGW), lambda i: (0, i, 0)),
        ],
        out_specs=pl.BlockSpec((RB, h), lambda i: (i, 0)),
        out_shape=jax.ShapeDtypeStruct((N_PAD, h), jnp.float32),
    )(x_pad, w, deg2)


def _comb_body(acc_ref, g_ref, deg_ref, out_ref):
    deg = deg_ref[0, :, 0] + deg_ref[1, :, 0] + 1.0
    dis = lax.rsqrt(deg)
    out_ref[...] = (acc_ref[0] + acc_ref[1] + g_ref[...]) * dis[:, None]


def _comb_call(acc2, g, deg2):
    h = acc2.shape[2]
    return pl.pallas_call(
        _comb_body,
        grid=(GRID,),
        in_specs=[
            pl.BlockSpec((NC, RB, h), lambda i: (0, i, 0)),
            pl.BlockSpec((RB, h), lambda i: (i, 0)),
            pl.BlockSpec((NC, RB, DEGW), lambda i: (0, i, 0)),
        ],
        out_specs=pl.BlockSpec((RB, h), lambda i: (i, 0)),
        out_shape=jax.ShapeDtypeStruct((N_PAD, h), jnp.float32),
    )(acc2, g, deg2)


def kernel(x, edge_index, W):
    n = x.shape[0]
    e = edge_index.shape[1]
    src = edge_index[0].astype(jnp.int32)
    dst = edge_index[1].astype(jnp.int32)

    ep = NS * (SA + SB) * B
    pad = ep - e
    ca = NS * SA * B
    src_f = jnp.concatenate([src, jnp.zeros((pad,), jnp.int32)])
    dst_f = jnp.concatenate([dst, jnp.full((pad,), DUMP, jnp.int32)])

    def split(flat):
        a = flat[:ca].reshape(NS, SA, B)
        b = jnp.pad(flat[ca:].reshape(NS, SB, B),
                    ((0, 0), (0, SA - SB), (0, 0)))
        return jnp.concatenate([a, b], axis=0)

    src_p = split(src_f)
    dst_p = split(dst_f)
    x_pad = jnp.pad(x, ((0, N_PAD - n), (0, 0)))

    deg2 = _deg_call(dst_p)
    g = _mm_call(x_pad, W, deg2)
    acc2 = _edge_call(src_p, dst_p, g)
    out_pad = _comb_call(acc2, g, deg2)
    return out_pad[:n]


# speedup vs baseline: 1.2346x; 1.2346x over previous
import jax
import jax.numpy as jnp
from jax import lax
from jax.experimental import pallas as pl
from jax.experimental.pallas import tpu as pltpu
from jax.experimental.pallas import tpu_sc as plsc

N_NODES = 10000
N_PAD = 10240
DUMP = 10000
NC, NS = 2, 16
NW = NC * NS
B = 64
SA = 160
SB = 160
SEG_STEPS = 32
SEG_ALL = SB // SEG_STEPS
SEG_A = SA // SEG_STEPS
NBUF = 4
LOOK = 2
DEGW = 16
ROWS_PER_TILE = N_PAD // NS
RB = 1024
GRID = N_PAD // RB


def _sc_mesh():
    return plsc.VectorSubcoreMesh(core_axis_name="c", subcore_axis_name="s")


def _deg_body(dst_hbm, deg_out, idx_v, ones_v, zb_v, deg_sh, dsem):
    cid = lax.axis_index("c")
    sid = lax.axis_index("s")
    wid = cid * NS + sid
    row0 = sid * ROWS_PER_TILE

    def fill_ones(t, c):
        ones_v[t, :] = jnp.full((DEGW,), 1.0, jnp.float32)
        return c

    lax.fori_loop(0, B, fill_ones, 0)

    def fill_z(t, c):
        zb_v[t, :] = jnp.zeros((DEGW,), jnp.float32)
        return c

    lax.fori_loop(0, 16, fill_z, 0)

    def zero_cp(t, c):
        pltpu.sync_copy(zb_v, deg_sh.at[pl.ds(row0 + t * 16, 16)])
        return c

    lax.fori_loop(0, ROWS_PER_TILE // 16, zero_cp, 0)
    plsc.subcore_barrier()

    pltpu.sync_copy(dst_hbm.at[wid], idx_v)

    def batch(tb, c):
        def fire(j, c2):
            pltpu.async_copy(ones_v, deg_sh.at[idx_v.at[tb * 8 + j]], dsem,
                             add=True)
            return c2

        lax.fori_loop(0, 8, fire, 0)

        def drain(j, c2):
            pltpu.make_async_copy(ones_v, deg_sh.at[idx_v.at[tb * 8 + j]],
                                  dsem).wait()
            return c2

        lax.fori_loop(0, 8, drain, 0)
        return c

    lax.fori_loop(0, SA // 8, batch, 0)
    plsc.subcore_barrier()

    pltpu.sync_copy(deg_sh.at[pl.ds(row0, ROWS_PER_TILE)],
                    deg_out.at[cid, pl.ds(row0, ROWS_PER_TILE)])


def _deg_call(dst_p):
    return pl.kernel(
        _deg_body,
        out_type=jax.ShapeDtypeStruct((NC, N_PAD, DEGW), jnp.float32),
        mesh=_sc_mesh(),
        scratch_types=[
            pltpu.VMEM((SA, B), jnp.int32),
            pltpu.VMEM((B, DEGW), jnp.float32),
            pltpu.VMEM((16, DEGW), jnp.float32),
            pltpu.VMEM_SHARED((N_PAD, DEGW), jnp.float32),
            pltpu.SemaphoreType.DMA,
        ],
    )(dst_p)


def _edge_body(src_hbm, dst_hbm, g_hbm, acc_out,
               src_v, dst_v, rows_v, zb_v, acc_sh,
               gsem0, gsem1, gsem2, gsem3, ssem0, ssem1, ssem2, ssem3):
    gsem = (gsem0, gsem1, gsem2, gsem3)
    ssem = (ssem0, ssem1, ssem2, ssem3)
    cid = lax.axis_index("c")
    sid = lax.axis_index("s")
    wid = cid * NS + sid
    row0 = sid * ROWS_PER_TILE

    def fill_z(t, c):
        r = t // 8
        col = t % 8
        zb_v[r, pl.ds(col * 16, 16)] = jnp.zeros((16,), jnp.float32)
        return c

    lax.fori_loop(0, 64, fill_z, 0)

    def zero_cp(t, c):
        pltpu.sync_copy(zb_v, acc_sh.at[pl.ds(row0 + t * 8, 8)])
        return c

    lax.fori_loop(0, ROWS_PER_TILE // 8, zero_cp, 0)
    plsc.subcore_barrier()

    def run_segment(s, active):
        def guard(extra, fn):
            def run():
                fn()

            if active is None and extra is None:
                run()
            elif active is None:
                pl.when(extra)(run)
            elif extra is None:
                pl.when(active)(run)
            else:
                pl.when(jnp.logical_and(active, extra))(run)

        guard(None, lambda: pltpu.sync_copy(
            src_hbm.at[wid, pl.ds(s * SEG_STEPS, SEG_STEPS)], src_v))
        guard(None, lambda: pltpu.sync_copy(
            dst_hbm.at[wid, pl.ds(s * SEG_STEPS, SEG_STEPS)], dst_v))

        for b in range(LOOK):
            guard(None, lambda b=b: pltpu.async_copy(
                g_hbm.at[src_v.at[b]], rows_v.at[b], gsem[b]))

        def outer(t, c):
            j0 = t * NBUF
            for b in range(NBUF):
                j = j0 + b
                b2 = (b + LOOK) % NBUF
                guard(None, lambda b=b, j=j: pltpu.make_async_copy(
                    g_hbm.at[src_v.at[j]], rows_v.at[b], gsem[b]).wait())
                guard(None, lambda b=b, j=j: pltpu.async_copy(
                    rows_v.at[b], acc_sh.at[dst_v.at[j]], ssem[b], add=True))
                guard(j >= LOOK, lambda b2=b2, j=j: pltpu.make_async_copy(
                    rows_v.at[b2], acc_sh.at[dst_v.at[j - LOOK]],
                    ssem[b2]).wait())
                guard(j + LOOK < SEG_STEPS, lambda b2=b2, j=j:
                      pltpu.async_copy(g_hbm.at[src_v.at[j + LOOK]],
                                       rows_v.at[b2], gsem[b2]))
            return c

        lax.fori_loop(0, SEG_STEPS // NBUF, outer, 0)
        for j in range(SEG_STEPS - LOOK, SEG_STEPS):
            guard(None, lambda j=j: pltpu.make_async_copy(
                rows_v.at[j % NBUF], acc_sh.at[dst_v.at[j]],
                ssem[j % NBUF]).wait())

    for s in range(SEG_A):
        run_segment(s, None if s < SEG_ALL else (cid == 0))
    plsc.subcore_barrier()

    pltpu.sync_copy(acc_sh.at[pl.ds(row0, ROWS_PER_TILE)],
                    acc_out.at[cid, pl.ds(row0, ROWS_PER_TILE)])


def _edge_call(src_p, dst_p, g):
    h = g.shape[1]
    return pl.kernel(
        _edge_body,
        out_type=jax.ShapeDtypeStruct((NC, N_PAD, h), jnp.float32),
        mesh=_sc_mesh(),
        scratch_types=[
            pltpu.VMEM((SEG_STEPS, B), jnp.int32),
            pltpu.VMEM((SEG_STEPS, B), jnp.int32),
            pltpu.VMEM((NBUF, B, h), jnp.float32),
            pltpu.VMEM((8, h), jnp.float32),
            pltpu.VMEM_SHARED((N_PAD, h), jnp.float32),
            pltpu.SemaphoreType.DMA,
            pltpu.SemaphoreType.DMA,
            pltpu.SemaphoreType.DMA,
            pltpu.SemaphoreType.DMA,
            pltpu.SemaphoreType.DMA,
            pltpu.SemaphoreType.DMA,
            pltpu.SemaphoreType.DMA,
            pltpu.SemaphoreType.DMA,
        ],
    )(src_p, dst_p, g)


def _mm_body(x_ref, w_ref, deg_ref, g_ref):
    deg = deg_ref[0, :, 0] + deg_ref[1, :, 0] + 1.0
    dis = lax.rsqrt(deg)
    h = jnp.dot(x_ref[...], w_ref[...], preferred_element_type=jnp.float32)
    g_ref[...] = h * dis[:, None]


def _mm_call(x_pad, w, deg2):
    c = x_pad.shape[1]
    h = w.shape[1]
    return pl.pallas_call(
        _mm_body,
        grid=(GRID,),
        in_specs=[
            pl.BlockSpec((RB, c), lambda i: (i, 0)),
            pl.BlockSpec((c, h), lambda i: (0, 0)),
            pl.BlockSpec((NC, RB, DEGW), lambda i: (0, i, 0)),
        ],
        out_specs=pl.BlockSpec((RB, h), lambda i: (i, 0)),
        out_shape=jax.ShapeDtypeStruct((N_PAD, h), jnp.float32),
    )(x_pad, w, deg2)


def _comb_body(acc_ref, g_ref, deg_ref, out_ref):
    deg = deg_ref[0, :, 0] + deg_ref[1, :, 0] + 1.0
    dis = lax.rsqrt(deg)
    out_ref[...] = (acc_ref[0] + acc_ref[1] + g_ref[...]) * dis[:, None]


def _comb_call(acc2, g, deg2):
    h = acc2.shape[2]
    return pl.pallas_call(
        _comb_body,
        grid=(GRID,),
        in_specs=[
            pl.BlockSpec((NC, RB, h), lambda i: (0, i, 0)),
            pl.BlockSpec((RB, h), lambda i: (i, 0)),
            pl.BlockSpec((NC, RB, DEGW), lambda i: (0, i, 0)),
        ],
        out_specs=pl.BlockSpec((RB, h), lambda i: (i, 0)),
        out_shape=jax.ShapeDtypeStruct((N_PAD, h), jnp.float32),
    )(acc2, g, deg2)


def kernel(x, edge_index, W):
    n = x.shape[0]
    e = edge_index.shape[1]
    src = edge_index[0].astype(jnp.int32)
    dst = edge_index[1].astype(jnp.int32)

    ep = NS * (SA + SB) * B
    pad = ep - e
    src_p = jnp.concatenate([src, jnp.zeros((pad,), jnp.int32)]).reshape(
        NW, SA, B)
    dst_p = jnp.concatenate([dst, jnp.full((pad,), DUMP, jnp.int32)]).reshape(
        NW, SA, B)
    x_pad = jnp.pad(x, ((0, N_PAD - n), (0, 0)))

    deg2 = _deg_call(dst_p)
    g = _mm_call(x_pad, W, deg2)
    acc2 = _edge_call(src_p, dst_p, g)
    out_pad = _comb_call(acc2, g, deg2)
    return out_pad[:n]
